# baseline (speedup 1.0000x reference)
import functools

import jax
import jax.numpy as jnp
from jax import lax
from jax.experimental import pallas as pl
from jax.experimental.pallas import tpu as pltpu
from jax.experimental.pallas import tpu_sc as plsc

B = 16384
S = 25
F = 128
E = 128
L = 16
NC, NS = 2, 16
NW = NC * NS
BPW = B // NW
CH = 4
NCHUNK = BPW // CH
NBUF = 4
SG = B // (NW * 128)


def _accum_chunk(nrows, obuf):
    for d in range(CH):
        r0 = d * S
        for j in range(F // L):
            sl = pl.ds(j * L, L)
            acc = nrows[r0, sl]
            for s in range(1, S):
                acc = acc + nrows[r0 + s, sl]
            obuf[d, sl] = acc


@functools.cache
def _build_sc_gather():
  mesh = plsc.VectorSubcoreMesh(core_axis_name="c", subcore_axis_name="s")

  @functools.partial(
    pl.kernel,
    out_type=[
        jax.ShapeDtypeStruct((B, F), jnp.float32),
        jax.ShapeDtypeStruct((B, F), jnp.float32),
    ],
    mesh=mesh,
    scratch_types=[
        pltpu.VMEM((SG, 128), jnp.int32),
        pltpu.VMEM((NCHUNK, CH * S), jnp.int32),
        pltpu.VMEM((2, 128, F), jnp.float32),
        [pltpu.VMEM((CH * S, F), jnp.float32)] * NBUF,
        [pltpu.VMEM((CH, F), jnp.float32)] * NBUF,
        [pltpu.SemaphoreType.DMA] * NBUF,
        [pltpu.SemaphoreType.DMA] * NBUF,
        [pltpu.SemaphoreType.DMA] * 2,
        [pltpu.SemaphoreType.DMA] * 2,
    ],
)
  def _sc_gather(nodes2, neigh2, feat, self_out, neigh_out,
                 nidx, eidx, srows, nrows, obufs, gsems, wsems, sgsems, swsems):
      wid = lax.axis_index("s") * NC + lax.axis_index("c")
      obase = wid * BPW

      pltpu.sync_copy(nodes2.at[pl.ds(wid * SG, SG)], nidx)
      pltpu.sync_copy(neigh2.at[pl.ds(wid * NCHUNK, NCHUNK)], eidx)

      for c in range(NBUF):
          pltpu.make_async_copy(feat.at[eidx.at[c]], nrows[c], gsems[c]).start()

      pltpu.make_async_copy(feat.at[nidx.at[0]], srows.at[0], sgsems[0]).start()
      pltpu.make_async_copy(feat.at[nidx.at[1]], srows.at[1], sgsems[1]).start()
      for g in range(SG):
          p = g % 2
          pltpu.make_async_copy(feat.at[nidx.at[g]], srows.at[p], sgsems[p]).wait()
          out_sl = self_out.at[pl.ds(obase + g * 128, 128)]
          pltpu.make_async_copy(srows.at[p], out_sl, swsems[p]).start()
          if g + 2 < SG:
              pltpu.make_async_copy(srows.at[p], out_sl, swsems[p]).wait()
              pltpu.make_async_copy(feat.at[nidx.at[g + 2]], srows.at[p], sgsems[p]).start()
      for g in range(SG - 2, SG):
          p = g % 2
          out_sl = self_out.at[pl.ds(obase + g * 128, 128)]
          pltpu.make_async_copy(srows.at[p], out_sl, swsems[p]).wait()

      def body(c2, carry):
          for k in range(NBUF):
              c = c2 * NBUF + k

              @pl.when(c >= NBUF)
              def _wait_write():
                  dst = neigh_out.at[pl.ds(obase + (c - NBUF) * CH, CH)]
                  pltpu.make_async_copy(obufs[k], dst, wsems[k]).wait()

              pltpu.make_async_copy(feat.at[eidx.at[c]], nrows[k], gsems[k]).wait()
              _accum_chunk(nrows[k], obufs[k])

              @pl.when(c + NBUF < NCHUNK)
              def _next_gather():
                  pltpu.make_async_copy(
                      feat.at[eidx.at[c + NBUF]], nrows[k], gsems[k]).start()

              dst = neigh_out.at[pl.ds(obase + c * CH, CH)]
              pltpu.make_async_copy(obufs[k], dst, wsems[k]).start()
          return carry

      lax.fori_loop(0, NCHUNK // NBUF, body, 0)

      for c in range(NCHUNK - NBUF, NCHUNK):
          k = c % NBUF
          dst = neigh_out.at[pl.ds(obase + c * CH, CH)]
          pltpu.make_async_copy(obufs[k], dst, wsems[k]).wait()

  return _sc_gather


def _tc_body(w_ref, s_ref, n_ref, o_ref):
    w1 = w_ref[:, :F]
    w2 = w_ref[:, F:]
    dn = (((1,), (1,)), ((), ()))
    acc = lax.dot_general(w1, s_ref[...], dn, preferred_element_type=jnp.float32)
    acc = acc + lax.dot_general(w2, n_ref[...], dn, preferred_element_type=jnp.float32)
    o_ref[...] = jnp.maximum(acc, 0.0)


_BLK = 2048


@jax.jit
def _tc_matmul(w, self_feats, neigh_sums):
    return pl.pallas_call(
        _tc_body,
        out_shape=jax.ShapeDtypeStruct((E, B), jnp.float32),
        grid=(B // _BLK,),
        in_specs=[
            pl.BlockSpec((E, 2 * F), lambda i: (0, 0)),
            pl.BlockSpec((_BLK, F), lambda i: (i, 0)),
            pl.BlockSpec((_BLK, F), lambda i: (i, 0)),
        ],
        out_specs=pl.BlockSpec((E, _BLK), lambda i: (0, i)),
    )(w, self_feats, neigh_sums)


def kernel(nodes, neigh_idx, features, weight):
    nodes2 = nodes.reshape(B // 128, 128).astype(jnp.int32)
    neigh2 = neigh_idx.reshape(B * S // (CH * S), CH * S).astype(jnp.int32)
    self_feats, neigh_sums = _build_sc_gather()(nodes2, neigh2, features)
    wscaled = jnp.concatenate([weight[:, :F], weight[:, F:] * (1.0 / S)], axis=1)
    return _tc_matmul(wscaled, self_feats, neigh_sums)


# speedup vs baseline: 1.1491x; 1.1491x over previous
import functools

import jax
import jax.numpy as jnp
from jax import lax
from jax.experimental import pallas as pl
from jax.experimental.pallas import tpu as pltpu
from jax.experimental.pallas import tpu_sc as plsc

B = 16384
S = 25
F = 128
E = 128
L = 16
NC, NS = 2, 16
NW = NC * NS
BPW = B // NW
CH = 4
NCHUNK = BPW // CH
NBUF = 2
SG = B // (NW * 128)


def _accum_chunk(nrows, obuf):
    for d in range(CH):
        r0 = d * S
        for j in range(F // L):
            sl = pl.ds(j * L, L)
            acc = nrows[r0, sl]
            for s in range(1, S):
                acc = acc + nrows[r0 + s, sl]
            obuf[d, sl] = acc


@functools.cache
def _build_sc_gather():
  mesh = plsc.VectorSubcoreMesh(core_axis_name="c", subcore_axis_name="s")

  @functools.partial(
    pl.kernel,
    out_type=[
        jax.ShapeDtypeStruct((B, F), jnp.float32),
        jax.ShapeDtypeStruct((B, F), jnp.float32),
    ],
    mesh=mesh,
    scratch_types=[
        pltpu.VMEM((SG, 128), jnp.int32),
        pltpu.VMEM((NCHUNK, CH * S), jnp.int32),
        pltpu.VMEM((2, 128, F), jnp.float32),
        [pltpu.VMEM((CH * S, F), jnp.float32)] * NBUF,
        [pltpu.VMEM((CH, F), jnp.float32)] * NBUF,
        [pltpu.SemaphoreType.DMA] * NBUF,
        [pltpu.SemaphoreType.DMA] * NBUF,
        [pltpu.SemaphoreType.DMA] * 2,
        [pltpu.SemaphoreType.DMA] * 2,
    ],
)
  def _sc_gather(nodes2, neigh2, feat, self_out, neigh_out,
                 nidx, eidx, srows, nrows, obufs, gsems, wsems, sgsems, swsems):
      wid = lax.axis_index("s") * NC + lax.axis_index("c")
      obase = wid * BPW

      pltpu.sync_copy(nodes2.at[pl.ds(wid * SG, SG)], nidx)
      pltpu.sync_copy(neigh2.at[pl.ds(wid * NCHUNK, NCHUNK)], eidx)

      for c in range(NBUF):
          pltpu.make_async_copy(feat.at[eidx.at[c]], nrows[c], gsems[c]).start()

      pltpu.make_async_copy(feat.at[nidx.at[0]], srows.at[0], sgsems[0]).start()
      pltpu.make_async_copy(feat.at[nidx.at[1]], srows.at[1], sgsems[1]).start()
      for g in range(SG):
          p = g % 2
          pltpu.make_async_copy(feat.at[nidx.at[g]], srows.at[p], sgsems[p]).wait()
          out_sl = self_out.at[pl.ds(obase + g * 128, 128)]
          pltpu.make_async_copy(srows.at[p], out_sl, swsems[p]).start()
          if g + 2 < SG:
              pltpu.make_async_copy(srows.at[p], out_sl, swsems[p]).wait()
              pltpu.make_async_copy(feat.at[nidx.at[g + 2]], srows.at[p], sgsems[p]).start()
      for g in range(SG - 2, SG):
          p = g % 2
          out_sl = self_out.at[pl.ds(obase + g * 128, 128)]
          pltpu.make_async_copy(srows.at[p], out_sl, swsems[p]).wait()

      def body(c2, carry):
          for k in range(NBUF):
              c = c2 * NBUF + k

              @pl.when(c >= NBUF)
              def _wait_write():
                  dst = neigh_out.at[pl.ds(obase + (c - NBUF) * CH, CH)]
                  pltpu.make_async_copy(obufs[k], dst, wsems[k]).wait()

              pltpu.make_async_copy(feat.at[eidx.at[c]], nrows[k], gsems[k]).wait()
              _accum_chunk(nrows[k], obufs[k])

              @pl.when(c + NBUF < NCHUNK)
              def _next_gather():
                  pltpu.make_async_copy(
                      feat.at[eidx.at[c + NBUF]], nrows[k], gsems[k]).start()

              dst = neigh_out.at[pl.ds(obase + c * CH, CH)]
              pltpu.make_async_copy(obufs[k], dst, wsems[k]).start()
          return carry

      lax.fori_loop(0, NCHUNK // NBUF, body, 0)

      for c in range(NCHUNK - NBUF, NCHUNK):
          k = c % NBUF
          dst = neigh_out.at[pl.ds(obase + c * CH, CH)]
          pltpu.make_async_copy(obufs[k], dst, wsems[k]).wait()

  return _sc_gather


def _tc_body(w_ref, s_ref, n_ref, o_ref):
    w1 = w_ref[:, :F]
    w2 = w_ref[:, F:]
    dn = (((1,), (1,)), ((), ()))
    acc = lax.dot_general(w1, s_ref[...], dn, preferred_element_type=jnp.float32)
    acc = acc + lax.dot_general(w2, n_ref[...], dn, preferred_element_type=jnp.float32)
    o_ref[...] = jnp.maximum(acc, 0.0)


_BLK = 2048


@jax.jit
def _tc_matmul(w, self_feats, neigh_sums):
    return pl.pallas_call(
        _tc_body,
        out_shape=jax.ShapeDtypeStruct((E, B), jnp.float32),
        grid=(B // _BLK,),
        in_specs=[
            pl.BlockSpec((E, 2 * F), lambda i: (0, 0)),
            pl.BlockSpec((_BLK, F), lambda i: (i, 0)),
            pl.BlockSpec((_BLK, F), lambda i: (i, 0)),
        ],
        out_specs=pl.BlockSpec((E, _BLK), lambda i: (0, i)),
    )(w, self_feats, neigh_sums)


def kernel(nodes, neigh_idx, features, weight):
    nodes2 = nodes.reshape(B // 128, 128).astype(jnp.int32)
    neigh2 = neigh_idx.reshape(B * S // (CH * S), CH * S).astype(jnp.int32)
    self_feats, neigh_sums = _build_sc_gather()(nodes2, neigh2, features)
    wscaled = jnp.concatenate([weight[:, :F], weight[:, F:] * (1.0 / S)], axis=1)
    return _tc_matmul(wscaled, self_feats, neigh_sums)
